# baseline (speedup 1.0000x reference)
import functools

import jax
import jax.numpy as jnp
from jax import lax
from jax.experimental import pallas as pl
from jax.experimental.pallas import tpu as pltpu
from jax.experimental.pallas import tpu_sc as plsc

U_NUM = 1000000
I_NUM = 100000
DIM = 16
B = 16384

_NC = 2
_NS = 16
_NW = _NC * _NS
_BPW = B // _NW
_NBLK = _BPW // 16
_CHUNK = 128


_mesh = plsc.VectorSubcoreMesh(core_axis_name="c", subcore_axis_name="s")


@functools.partial(
    pl.kernel,
    out_type=jax.ShapeDtypeStruct((B,), jnp.float32),
    mesh=_mesh,
    scratch_types=[
        pltpu.VMEM((_BPW,), jnp.int32),
        pltpu.VMEM((_BPW,), jnp.int32),
        pltpu.VMEM((_BPW,), jnp.int32),
        pltpu.VMEM((_BPW, DIM), jnp.float32),
        pltpu.VMEM((_BPW, DIM), jnp.float32),
        pltpu.VMEM((_BPW, DIM), jnp.float32),
        pltpu.VMEM((_BPW, DIM), jnp.float32),
        pltpu.VMEM((_BPW,), jnp.float32),
        pltpu.SemaphoreType.DMA,
    ],
    compiler_params=pltpu.CompilerParams(
        needs_layout_passes=False, use_tc_tiling_on_sc=False
    ),
)
def _irt_sc(uid_hbm, iid_hbm, theta_hbm, a_hbm, b_hbm, c_hbm, out_hbm,
            uid_v, iid_v, cidx_v, th_v, a_v, b_v, c_v, out_v, sem):
    wid = lax.axis_index("s") * _NC + lax.axis_index("c")
    base = wid * _BPW

    pltpu.sync_copy(uid_hbm.at[pl.ds(base, _BPW)], uid_v)
    pltpu.sync_copy(iid_hbm.at[pl.ds(base, _BPW)], iid_v)

    cps = []
    for k in range(_BPW // _CHUNK):
        sl = pl.ds(k * _CHUNK, _CHUNK)
        u_k = uid_v.at[sl]
        i_k = iid_v.at[sl]
        cps.append(pltpu.async_copy(theta_hbm.at[u_k], th_v.at[sl, :], sem))
        cps.append(pltpu.async_copy(a_hbm.at[i_k], a_v.at[sl, :], sem))
        cps.append(pltpu.async_copy(b_hbm.at[i_k], b_v.at[sl, :], sem))

    def cidx_body(j, _):
        s = pl.ds(j * 16, 16)
        cidx_v[s] = lax.shift_right_logical(iid_v[s], 4)
        return _
    lax.fori_loop(0, _NBLK, cidx_body, 0, unroll=False)
    for k in range(_BPW // _CHUNK):
        sl = pl.ds(k * _CHUNK, _CHUNK)
        cps.append(pltpu.async_copy(c_hbm.at[cidx_v.at[sl]], c_v.at[sl, :], sem))
    for cp in cps:
        cp.wait()

    lane = lax.iota(jnp.int32, 16)
    dcoef = jnp.full((16,), 1.702, jnp.float32)
    one = jnp.full((16,), 1.0, jnp.float32)

    def blk_body(blk, _):
        rows = lane + blk * 16
        acc = jnp.zeros((16,), jnp.float32)
        for t in range(DIM):
            d_idx = (lane + t) & 15
            th = plsc.load_gather(th_v, [rows, d_idx])
            av = plsc.load_gather(a_v, [rows, d_idx])
            bv = plsc.load_gather(b_v, [rows, d_idx])
            acc = acc + av * (th - bv)
        craw = plsc.load_gather(c_v, [rows, iid_v[pl.ds(blk * 16, 16)] & 15])
        cs = one / (one + jnp.exp(-craw))
        sig = one / (one + jnp.exp(-dcoef * acc))
        out_v[pl.ds(blk * 16, 16)] = cs + (one - cs) * sig
        return _

    lax.fori_loop(0, _NBLK, blk_body, 0, unroll=False)

    pltpu.sync_copy(out_v, out_hbm.at[pl.ds(base, _BPW)])


def kernel(user_id, item_id, theta_w, a_w, b_w, c_w):
    uid = jnp.asarray(user_id, jnp.int32)
    iid = jnp.asarray(item_id, jnp.int32)
    c_t = jnp.reshape(c_w, (I_NUM // DIM, DIM))
    return _irt_sc(uid, iid, theta_w, a_w, b_w, c_t)
